# baseline (speedup 1.0000x reference)
import functools

import jax
import jax.numpy as jnp
from jax import lax
from jax.experimental import pallas as pl
from jax.experimental.pallas import tpu as pltpu
from jax.experimental.pallas import tpu_sc as plsc

_NUM_WORKERS = 32
_G = 64


def kernel(x, table):
    batch, fields = x.shape
    depth = table.shape[1]
    assert batch % (_NUM_WORKERS * _G) == 0
    rows_w = batch // _NUM_WORKERS
    num_groups = rows_w // _G

    xi = x.astype(jnp.int32)

    mesh = plsc.VectorSubcoreMesh(core_axis_name="c", subcore_axis_name="s")

    @functools.partial(
        pl.kernel,
        mesh=mesh,
        compiler_params=pltpu.CompilerParams(use_tc_tiling_on_sc=False),
        out_type=jax.ShapeDtypeStruct((batch, fields, depth), jnp.float32),
        scratch_types=[
            pltpu.VMEM((batch // _NUM_WORKERS, fields), jnp.int32),
            pltpu.VMEM((2 * _G, fields, depth), jnp.float32),
            pltpu.SemaphoreType.DMA,
            pltpu.SemaphoreType.DMA,
            pltpu.SemaphoreType.DMA,
        ],
    )
    def body(x_hbm, table_hbm, out_hbm, idx_v, buf, sem_g, sem_w0, sem_w1):
        wid = lax.axis_index("s") * 2 + lax.axis_index("c")
        b0 = wid * rows_w
        pltpu.sync_copy(x_hbm.at[pl.ds(b0, rows_w)], idx_v)

        def out_slice(t):
            return out_hbm.at[pl.ds(b0 + t * _G, _G)]

        def buf_slot(t):
            return buf.at[pl.ds((t % 2) * _G, _G)]

        def write_group(t, sem):
            return pltpu.make_async_copy(buf_slot(t), out_slice(t), sem)

        def step(t, carry):
            slot = (t % 2) * _G
            parity = t % 2

            @pl.when(t >= 2)
            def _():
                @pl.when(parity == 0)
                def _():
                    write_group(t - 2, sem_w0).wait()

                @pl.when(parity == 1)
                def _():
                    write_group(t - 2, sem_w1).wait()

            def fire(r, c):
                pltpu.async_copy(
                    table_hbm.at[idx_v.at[t * _G + r]],
                    buf.at[slot + r],
                    sem_g,
                )
                return c

            lax.fori_loop(0, _G, fire, 0)
            pltpu.make_async_copy(out_slice(t), buf_slot(t), sem_g).wait()

            @pl.when(parity == 0)
            def _():
                write_group(t, sem_w0).start()

            @pl.when(parity == 1)
            def _():
                write_group(t, sem_w1).start()

            return carry

        lax.fori_loop(0, num_groups, step, 0)
        if num_groups % 2 == 0:
            write_group(num_groups - 2, sem_w0).wait()
            write_group(num_groups - 1, sem_w1).wait()
        else:
            write_group(num_groups - 2, sem_w1).wait()
            write_group(num_groups - 1, sem_w0).wait()

    return body(xi, table)
